# baseline (speedup 1.0000x reference)
import functools

import jax
import jax.numpy as jnp
from jax import lax
from jax.experimental import pallas as pl
from jax.experimental.pallas import tpu as pltpu, tpu_sc as plsc

B = 4096
S = 200
D = 64
DP = 128
NC = 2
NS = 16
NW = NC * NS
B_PER_W = B // NW
NB = 4
GROUPS = ((0, 104), (104, 96))

_mesh = plsc.VectorSubcoreMesh(
    core_axis_name="c", subcore_axis_name="s", num_cores=NC, num_subcores=NS
)


@functools.partial(
    pl.kernel,
    out_type=jax.ShapeDtypeStruct((B, S, DP), jnp.float32),
    mesh=_mesh,
    compiler_params=pltpu.CompilerParams(use_tc_tiling_on_sc=False),
    scratch_types=[
        pltpu.VMEM((S * D,), jnp.float32),
        pltpu.VMEM((B_PER_W * S,), jnp.int32),
        pltpu.VMEM((NB, S, D), jnp.float32),
        [pltpu.SemaphoreType.DMA] * NB,
        [pltpu.SemaphoreType.DMA] * NB,
    ],
)
def _embed_sc(idx_hbm, emb_hbm, pos_hbm, out_hbm, pos_v, idx_v, rows_v,
              gsems, ssems):
    wid = lax.axis_index("s") * NC + lax.axis_index("c")
    b_base = wid * B_PER_W

    pltpu.sync_copy(idx_hbm.at[pl.ds(b_base * S, B_PER_W * S)], idx_v)
    pltpu.sync_copy(pos_hbm, pos_v)

    def out_slice(c):
        return out_hbm.at[b_base + c, slice(None), pl.ds(0, D)]

    def start_gather(c, buf):
        for off, n in GROUPS:
            pltpu.async_copy(
                emb_hbm.at[idx_v.at[pl.ds(c * S + off, n)]],
                rows_v.at[buf, pl.ds(off, n)],
                gsems[buf],
            )

    def drain_gather(buf):
        for off, n in GROUPS:
            pltpu.make_async_copy(
                emb_hbm.at[idx_v.at[pl.ds(0, n)]],
                rows_v.at[buf, pl.ds(off, n)],
                gsems[buf],
            ).wait()

    def add_pos(buf):
        def add_rows(s, _):
            for d in range(0, D, 16):
                plsc.addupdate(
                    rows_v.at[buf, s, pl.ds(d, 16)],
                    pos_v[pl.ds(s * D + d, 16)],
                )
            return 0

        lax.fori_loop(0, S, add_rows, 0, unroll=4)

    def start_store(c, buf):
        pltpu.async_copy(rows_v.at[buf], out_slice(c), ssems[buf])

    def drain_store(buf):
        pltpu.make_async_copy(rows_v.at[buf], out_slice(0), ssems[buf]).wait()

    start_gather(0, 0)
    for k in range(NB):
        nxt = (k + 1) % NB
        if k == NB - 1:
            drain_store(nxt)
        start_gather(k + 1, nxt)
        drain_gather(k)
        add_pos(k)
        start_store(k, k)

    def macro(m, _):
        c0 = m * NB
        for k in range(NB):
            c = c0 + k
            nxt = (k + 1) % NB
            drain_store(nxt)
            start_gather(c + 1, nxt)
            drain_gather(k)
            add_pos(k)
            start_store(c, k)
        return 0

    lax.fori_loop(1, B_PER_W // NB - 1, macro, 0)

    for k in range(NB):
        c = B_PER_W - NB + k
        if k < NB - 1:
            nxt = (k + 1) % NB
            drain_store(nxt)
            start_gather(c + 1, nxt)
        drain_gather(k)
        add_pos(k)
        start_store(c, k)

    for k in range(NB):
        drain_store(k)


def kernel(instruction, emb_table, pos_table):
    idx = instruction.reshape(-1).astype(jnp.int32)
    pos = pos_table[:S].reshape(-1)
    out = _embed_sc(idx, emb_table, pos)
    return out[..., :D]
